# Initial kernel scaffold; baseline (speedup 1.0000x reference)
import jax
import jax.numpy as jnp
from jax.experimental import pallas as pl


def kernel(x, edge_index, edge_attr, u, batch, e_w1, e_b1, e_w2, e_b2, ln_g, ln_b, Mk, Mv, n_w1, n_b1, n_w2, n_b2, g_w1, g_b1, g_w2, g_b2):
    raise NotImplementedError("write your pallas kernel here")


# speedup vs baseline: 5.1995x; 5.1995x over previous
import functools

import jax
import jax.numpy as jnp
from jax import lax
from jax.experimental import pallas as pl
from jax.experimental.pallas import tpu as pltpu
from jax.experimental.pallas import tpu_sc as plsc

N = 10000
E = 320000
B = 256
V = 128
EIN = 16
U = 3
H = 128
D_NODE = V + H + U
D_PAD = 384

F32 = jnp.float32
_DN0 = (((0,), (0,)), ((), ()))


def _dot(a, b):
    return jnp.dot(a, b, preferred_element_type=F32)


def _pre_body(x_ref, up_ref, b2d_ref, w1aT_ref, w1bT_ref, w1dT_ref, b1_ref,
              xa_ref, xb_ref):
    ua = _dot(up_ref[...], w1dT_ref[...])
    oh = (b2d_ref[...] == lax.broadcasted_iota(jnp.int32, (N, B), 1)
          ).astype(F32)
    ubn = _dot(oh, ua)
    xa_ref[...] = _dot(x_ref[...], w1aT_ref[...]) + ubn + b1_ref[...]
    xb_ref[...] = _dot(x_ref[...], w1bT_ref[...])


def _precompute(x, u_pad, batch2d, w1aT, w1bT, w1dT_pad, b1row):
    return pl.pallas_call(
        _pre_body,
        out_shape=(jax.ShapeDtypeStruct((N, H), F32),
                   jax.ShapeDtypeStruct((N, H), F32)),
    )(x, u_pad, batch2d, w1aT, w1bT, w1dT_pad, b1row)


_NC = 2
_NS = 16
_NW = _NC * _NS
_EW = E // _NW
_CH = 80
_NCH = _EW // _CH


def _gather_body(xa_hbm, xb_hbm, row_hbm, col_hbm, xr_hbm, xc_hbm,
                 idxr, idxc, bufa, bufb, sema, semb):
    wid = lax.axis_index("s") * _NC + lax.axis_index("c")

    def chunk(i, carry):
        base = wid * _EW + i * _CH
        pltpu.sync_copy(row_hbm.at[pl.ds(base, _CH)], idxr)
        pltpu.sync_copy(col_hbm.at[pl.ds(base, _CH)], idxc)
        ca = pltpu.async_copy(xa_hbm.at[idxr], bufa, sema)
        cb = pltpu.async_copy(xb_hbm.at[idxc], bufb, semb)
        ca.wait()
        cb.wait()
        pltpu.sync_copy(bufa, xr_hbm.at[pl.ds(base, _CH)])
        pltpu.sync_copy(bufb, xc_hbm.at[pl.ds(base, _CH)])
        return carry

    lax.fori_loop(0, _NCH, chunk, 0)


def _sc_gather(xa2, xb2, row, col):
    mesh = plsc.VectorSubcoreMesh(core_axis_name="c", subcore_axis_name="s")
    fn = pl.kernel(
        _gather_body,
        out_type=(jax.ShapeDtypeStruct((E, H), F32),
                  jax.ShapeDtypeStruct((E, H), F32)),
        mesh=mesh,
        scratch_types=[
            pltpu.VMEM((_CH,), jnp.int32),
            pltpu.VMEM((_CH,), jnp.int32),
            pltpu.VMEM((_CH, H), F32),
            pltpu.VMEM((_CH, H), F32),
            pltpu.SemaphoreType.DMA,
            pltpu.SemaphoreType.DMA,
        ],
    )
    return fn(xa2, xb2, row, col)


_BE = 2000
_NBE = E // _BE


def _edge_body(xr_ref, xc_ref, ea_ref, w1cT_ref, w2T_ref, b2_ref, out_ref):
    pre = xr_ref[...] + xc_ref[...] + _dot(ea_ref[...], w1cT_ref[...])
    h = jnp.maximum(pre, 0.0)
    out_ref[...] = _dot(h, w2T_ref[...]) + b2_ref[...]


def _edge_mlp(xr, xc, edge_attr, w1cT, w2T, b2row):
    const = pl.BlockSpec((1, H), lambda i: (0, 0))
    return pl.pallas_call(
        _edge_body,
        grid=(_NBE,),
        in_specs=[
            pl.BlockSpec((_BE, H), lambda i: (i, 0)),
            pl.BlockSpec((_BE, H), lambda i: (i, 0)),
            pl.BlockSpec((_BE, EIN), lambda i: (i, 0)),
            pl.BlockSpec((EIN, H), lambda i: (0, 0)),
            pl.BlockSpec((H, H), lambda i: (0, 0)),
            const,
        ],
        out_specs=pl.BlockSpec((_BE, H), lambda i: (i, 0)),
        out_shape=jax.ShapeDtypeStruct((E, H), F32),
    )(xr, xc, edge_attr, w1cT, w2T, b2row)


_ES2 = E // _NS
_NCH2 = _ES2 // _CH
_NR = N // _NC
_NRP = 5200
_ZCH = 200
_NZ = _NRP // _ZCH
_NW0 = _NR // _ZCH


def _scatter_body(en_hbm, col_hbm, agg_hbm, deg_hbm, idx, rows, stage, acc):
    c = lax.axis_index("c")
    s = lax.axis_index("s")
    lo = c * _NR
    z16 = jnp.zeros((16,), F32)
    o16 = jnp.full((16,), 1.0, F32)

    def _fill_stage(val):
        def zrow(r, carry):
            for j in range(8):
                stage[r, pl.ds(16 * j, 16)] = val
            return carry
        lax.fori_loop(0, _ZCH, zrow, 0)

    def _zero_acc():
        for k in range(-(-_NZ // _NS)):
            j = s + k * _NS

            @pl.when(j < _NZ)
            def _():
                pltpu.sync_copy(stage, acc.at[pl.ds(j * _ZCH, _ZCH)])

    def _scatter(load_rows):
        def chunk(i, carry):
            base = s * _ES2 + i * _CH
            pltpu.sync_copy(col_hbm.at[pl.ds(base, _CH)], idx)
            if load_rows:
                pltpu.sync_copy(en_hbm.at[pl.ds(base, _CH)], rows)
            for j in range(_CH // 16):
                v = idx[pl.ds(16 * j, 16)] - lo
                ok = (v >= 0) & (v < _NR)
                idx[pl.ds(16 * j, 16)] = jnp.where(ok, v, _NR)
            pltpu.sync_copy(rows, acc.at[idx], add=True)
            return carry
        lax.fori_loop(0, _NCH2, chunk, 0)

    def _writeout(dst_hbm):
        for k in range(-(-_NW0 // _NS)):
            j = s + k * _NS

            @pl.when(j < _NW0)
            def _():
                pltpu.sync_copy(acc.at[pl.ds(j * _ZCH, _ZCH)], stage)
                pltpu.sync_copy(stage, dst_hbm.at[pl.ds(lo + j * _ZCH, _ZCH)])

    _fill_stage(z16)
    _zero_acc()
    plsc.subcore_barrier()
    _scatter(load_rows=True)
    plsc.subcore_barrier()
    _writeout(agg_hbm)
    plsc.subcore_barrier()

    _fill_stage(z16)
    _zero_acc()

    def orow(r, carry):
        for j in range(8):
            rows[r, pl.ds(16 * j, 16)] = o16
        return carry
    lax.fori_loop(0, _CH, orow, 0)
    plsc.subcore_barrier()
    _scatter(load_rows=False)
    plsc.subcore_barrier()
    _writeout(deg_hbm)


def _sc_scatter(e_new, col):
    mesh = plsc.VectorSubcoreMesh(core_axis_name="c", subcore_axis_name="s")
    fn = pl.kernel(
        _scatter_body,
        out_type=(jax.ShapeDtypeStruct((N, H), F32),
                  jax.ShapeDtypeStruct((N, H), F32)),
        mesh=mesh,
        scratch_types=[
            pltpu.VMEM((_CH,), jnp.int32),
            pltpu.VMEM((_CH, H), F32),
            pltpu.VMEM((_ZCH, H), F32),
            pltpu.VMEM_SHARED((_NRP, H), F32),
        ],
    )
    return fn(e_new, col)


_BN = 2000
_NBN = N // _BN


def _node_body(x_ref, agg_ref, deg_ref, b2d_ref, up_ref,
               lng_ref, lnb_ref, mkT_ref, mv_ref,
               nw1T_ref, nb1_ref, nw2T_ref, nb2_ref,
               g1uT_ref, g1nT_ref, g1eT_ref, gb1_ref, gw2T_ref, gb2_ref,
               xn_ref, un_ref,
               ns_ref, nc_ref, es_ref, ec_ref):
    i = pl.program_id(0)

    @pl.when(i == 0)
    def _():
        ns_ref[...] = jnp.zeros_like(ns_ref)
        nc_ref[...] = jnp.zeros_like(nc_ref)
        es_ref[...] = jnp.zeros_like(es_ref)
        ec_ref[...] = jnp.zeros_like(ec_ref)

    xb = x_ref[...]
    aggb = agg_ref[...]
    degb = deg_ref[...][:, 0:1]
    oh = (b2d_ref[...] == lax.broadcasted_iota(jnp.int32, (_BN, B), 1)
          ).astype(F32)
    ub = _dot(oh, up_ref[...])
    out = jnp.concatenate([xb, aggb, ub], axis=1)
    s1 = jnp.sum(out, axis=1, keepdims=True)
    mu = s1 * (1.0 / D_NODE)
    s2 = jnp.sum(out * out, axis=1, keepdims=True)
    var = s2 * (1.0 / D_NODE) - mu * mu
    r = lax.rsqrt(var + 1e-5)
    outn = (out - mu) * r * lng_ref[...] + lnb_ref[...]
    sc = _dot(outn, mkT_ref[...])
    m = jnp.max(sc, axis=1, keepdims=True)
    ex = jnp.exp(sc - m)
    p = ex / jnp.sum(ex, axis=1, keepdims=True)
    attn = _dot(p, mv_ref[...])
    out2 = 0.5 * attn + 0.5 * outn
    hn = jnp.maximum(_dot(out2, nw1T_ref[...]) + nb1_ref[...], 0.0)
    xn = _dot(hn, nw2T_ref[...]) + nb2_ref[...]
    xn_ref[...] = xn

    ones = jnp.ones((_BN, H), F32)
    degm = jnp.broadcast_to(degb, (_BN, H))
    ns_ref[...] += lax.dot_general(oh, xn, _DN0, preferred_element_type=F32)
    nc_ref[...] += lax.dot_general(oh, ones, _DN0, preferred_element_type=F32)
    es_ref[...] += lax.dot_general(oh, aggb, _DN0, preferred_element_type=F32)
    ec_ref[...] += lax.dot_general(oh, degm, _DN0, preferred_element_type=F32)

    @pl.when(i == _NBN - 1)
    def _():
        na = ns_ref[...] / jnp.maximum(nc_ref[...], 1.0)
        eg = es_ref[...] / jnp.maximum(ec_ref[...], 1.0)
        hg = jnp.maximum(_dot(up_ref[...], g1uT_ref[...]) + _dot(na, g1nT_ref[...])
                         + _dot(eg, g1eT_ref[...]) + gb1_ref[...], 0.0)
        un_ref[...] = _dot(hg, gw2T_ref[...]) + gb2_ref[...]


def _node_global(x, agg, deg, batch2d, u_pad, lng_pad, lnb_pad, mkT_pad,
                 mv_pad, nw1T_pad, nb1row, nw2T, nb2row,
                 g1uT_pad, g1nT, g1eT, gb1row, gw2T, gb2row):
    cw = lambda shape: pl.BlockSpec(shape, lambda i: tuple(0 for _ in shape))
    return pl.pallas_call(
        _node_body,
        grid=(_NBN,),
        in_specs=[
            pl.BlockSpec((_BN, H), lambda i: (i, 0)),
            pl.BlockSpec((_BN, H), lambda i: (i, 0)),
            pl.BlockSpec((_BN, H), lambda i: (i, 0)),
            pl.BlockSpec((_BN, 1), lambda i: (i, 0)),
            cw((B, H)),
            cw((1, D_PAD)), cw((1, D_PAD)),
            cw((D_PAD, H)), cw((H, D_PAD)),
            cw((D_PAD, H)), cw((1, H)), cw((H, H)), cw((1, H)),
            cw((H, H)), cw((H, H)), cw((H, H)), cw((1, H)),
            cw((H, H)), cw((1, H)),
        ],
        out_specs=(
            pl.BlockSpec((_BN, H), lambda i: (i, 0)),
            pl.BlockSpec((B, H), lambda i: (0, 0)),
        ),
        out_shape=(jax.ShapeDtypeStruct((N, H), F32),
                   jax.ShapeDtypeStruct((B, H), F32)),
        scratch_shapes=[
            pltpu.VMEM((B, H), F32),
            pltpu.VMEM((B, H), F32),
            pltpu.VMEM((B, H), F32),
            pltpu.VMEM((B, H), F32),
        ],
    )(x, agg, deg, batch2d, u_pad, lng_pad, lnb_pad, mkT_pad, mv_pad,
      nw1T_pad, nb1row, nw2T, nb2row, g1uT_pad, g1nT, g1eT, gb1row, gw2T,
      gb2row)


def kernel(x, edge_index, edge_attr, u, batch,
           e_w1, e_b1, e_w2, e_b2, ln_g, ln_b, Mk, Mv,
           n_w1, n_b1, n_w2, n_b2, g_w1, g_b1, g_w2, g_b2):
    row = edge_index[0]
    col = edge_index[1]
    batch2d = batch.reshape(N, 1)

    w1aT = e_w1[:, :V].T
    w1bT = e_w1[:, V:2 * V].T
    w1cT = e_w1[:, 2 * V:2 * V + EIN].T
    w1dT_pad = jnp.zeros((H, H), F32).at[:U].set(e_w1[:, 2 * V + EIN:].T)
    u_pad = jnp.zeros((B, H), F32).at[:, :U].set(u)
    b1row = e_b1.reshape(1, H)
    w2T = e_w2.T
    b2row = e_b2.reshape(1, H)

    lng_pad = jnp.zeros((1, D_PAD), F32).at[0, :D_NODE].set(ln_g)
    lnb_pad = jnp.zeros((1, D_PAD), F32).at[0, :D_NODE].set(ln_b)
    mkT_pad = jnp.zeros((D_PAD, H), F32).at[:D_NODE].set(Mk.T)
    mv_pad = jnp.zeros((H, D_PAD), F32).at[:, :D_NODE].set(Mv)
    nw1T_pad = jnp.zeros((D_PAD, H), F32).at[:D_NODE].set(n_w1.T)
    nb1row = n_b1.reshape(1, H)
    nw2T = n_w2.T
    nb2row = n_b2.reshape(1, H)
    g1uT_pad = jnp.zeros((H, H), F32).at[:U].set(g_w1[:, :U].T)
    g1nT = g_w1[:, U:U + H].T
    g1eT = g_w1[:, U + H:].T
    gb1row = g_b1.reshape(1, H)
    gw2T = g_w2.T
    gb2row = g_b2.reshape(1, H)

    xa2, xb2 = _precompute(x, u_pad, batch2d, w1aT, w1bT, w1dT_pad, b1row)
    xr, xc = _sc_gather(xa2, xb2, row, col)
    e_new = _edge_mlp(xr, xc, edge_attr, w1cT, w2T, b2row)
    agg, deg = _sc_scatter(e_new, col)
    x_new, u_new = _node_global(
        x, agg, deg, batch2d, u_pad,
        lng_pad, lnb_pad, mkT_pad, mv_pad, nw1T_pad, nb1row, nw2T, nb2row,
        g1uT_pad, g1nT, g1eT, gb1row, gw2T, gb2row)
    return (x_new, e_new, u_new)


# speedup vs baseline: 5.9059x; 1.1359x over previous
import functools

import jax
import jax.numpy as jnp
from jax import lax
from jax.experimental import pallas as pl
from jax.experimental.pallas import tpu as pltpu
from jax.experimental.pallas import tpu_sc as plsc

N = 10000
E = 320000
B = 256
V = 128
EIN = 16
U = 3
H = 128
D_NODE = V + H + U
D_PAD = 384

F32 = jnp.float32
_DN0 = (((0,), (0,)), ((), ()))


def _dot(a, b):
    return jnp.dot(a, b, preferred_element_type=F32)


def _pre_body(x_ref, up_ref, b2d_ref, w1aT_ref, w1bT_ref, w1dT_ref, b1_ref,
              xa_ref, xb_ref, st_ref):
    ua = _dot(up_ref[...], w1dT_ref[...])
    giota = lax.broadcasted_iota(jnp.int32, (N, B), 1)
    oh = (b2d_ref[...] == giota).astype(F32)
    ubn = _dot(oh, ua)
    xa_ref[...] = _dot(x_ref[...], w1aT_ref[...]) + ubn + b1_ref[...]
    xb_ref[...] = _dot(x_ref[...], w1bT_ref[...])
    lt = (b2d_ref[...] < giota).astype(F32)
    st_ref[...] = _dot(jnp.ones((1, N), F32), lt)


def _precompute(x, u_pad, batch2d, w1aT, w1bT, w1dT_pad, b1row):
    return pl.pallas_call(
        _pre_body,
        out_shape=(jax.ShapeDtypeStruct((N, H), F32),
                   jax.ShapeDtypeStruct((N, H), F32),
                   jax.ShapeDtypeStruct((1, B), F32)),
    )(x, u_pad, batch2d, w1aT, w1bT, w1dT_pad, b1row)


_NC = 2
_NS = 16
_NW = _NC * _NS
_EW = E // _NW
_CH = 80
_NCH = _EW // _CH


def _gather_body(xa_hbm, xb_hbm, row_hbm, col_hbm, xr_hbm, xc_hbm,
                 idxr, idxc, bufa, bufb, sema, semb):
    wid = lax.axis_index("s") * _NC + lax.axis_index("c")

    def chunk(i, carry):
        base = wid * _EW + i * _CH
        pltpu.sync_copy(row_hbm.at[pl.ds(base, _CH)], idxr)
        pltpu.sync_copy(col_hbm.at[pl.ds(base, _CH)], idxc)
        ca = pltpu.async_copy(xa_hbm.at[idxr], bufa, sema)
        cb = pltpu.async_copy(xb_hbm.at[idxc], bufb, semb)
        ca.wait()
        cb.wait()
        pltpu.sync_copy(bufa, xr_hbm.at[pl.ds(base, _CH)])
        pltpu.sync_copy(bufb, xc_hbm.at[pl.ds(base, _CH)])
        return carry

    lax.fori_loop(0, _NCH, chunk, 0)


def _sc_gather(xa2, xb2, row, col):
    mesh = plsc.VectorSubcoreMesh(core_axis_name="c", subcore_axis_name="s")
    fn = pl.kernel(
        _gather_body,
        out_type=(jax.ShapeDtypeStruct((E, H), F32),
                  jax.ShapeDtypeStruct((E, H), F32)),
        mesh=mesh,
        scratch_types=[
            pltpu.VMEM((_CH,), jnp.int32),
            pltpu.VMEM((_CH,), jnp.int32),
            pltpu.VMEM((_CH, H), F32),
            pltpu.VMEM((_CH, H), F32),
            pltpu.SemaphoreType.DMA,
            pltpu.SemaphoreType.DMA,
        ],
    )
    return fn(xa2, xb2, row, col)


_BE = 2000
_NBE = E // _BE


def _edge_body(xr_ref, xc_ref, ea_ref, col_ref, st_ref, w1cT_ref, w2T_ref,
               b2_ref, out_ref, cum_ref):
    i = pl.program_id(0)

    @pl.when(i == 0)
    def _():
        cum_ref[...] = jnp.zeros_like(cum_ref)

    pre = xr_ref[...] + xc_ref[...] + _dot(ea_ref[...], w1cT_ref[...])
    h = jnp.maximum(pre, 0.0)
    out_ref[...] = _dot(h, w2T_ref[...]) + b2_ref[...]

    colf = col_ref[...].astype(F32)
    ge = (colf >= st_ref[...]).astype(F32)
    cum_ref[...] += _dot(jnp.ones((1, _BE), F32), ge)


def _edge_mlp(xr, xc, edge_attr, col2d, starts, w1cT, w2T, b2row):
    const = pl.BlockSpec((1, H), lambda i: (0, 0))
    return pl.pallas_call(
        _edge_body,
        grid=(_NBE,),
        in_specs=[
            pl.BlockSpec((_BE, H), lambda i: (i, 0)),
            pl.BlockSpec((_BE, H), lambda i: (i, 0)),
            pl.BlockSpec((_BE, EIN), lambda i: (i, 0)),
            pl.BlockSpec((_BE, 1), lambda i: (i, 0)),
            pl.BlockSpec((1, B), lambda i: (0, 0)),
            pl.BlockSpec((EIN, H), lambda i: (0, 0)),
            pl.BlockSpec((H, H), lambda i: (0, 0)),
            const,
        ],
        out_specs=(pl.BlockSpec((_BE, H), lambda i: (i, 0)),
                   pl.BlockSpec((1, B), lambda i: (0, 0))),
        out_shape=(jax.ShapeDtypeStruct((E, H), F32),
                   jax.ShapeDtypeStruct((1, B), F32)),
    )(xr, xc, edge_attr, col2d, starts, w1cT, w2T, b2row)


_ES2 = E // _NS
_NCH2 = _ES2 // _CH
_NR = N // _NC
_NRP = 5200
_ZCH = 200
_NZ = _NRP // _ZCH
_NW0 = _NR // _ZCH


def _scatter_body(en_hbm, col_hbm, agg_hbm, idx, rows, stage, acc):
    c = lax.axis_index("c")
    s = lax.axis_index("s")
    lo = c * _NR
    z16 = jnp.zeros((16,), F32)

    def _fill_stage(val):
        def zrow(r, carry):
            for j in range(8):
                stage[r, pl.ds(16 * j, 16)] = val
            return carry
        lax.fori_loop(0, _ZCH, zrow, 0)

    def _zero_acc():
        for k in range(-(-_NZ // _NS)):
            j = s + k * _NS

            @pl.when(j < _NZ)
            def _():
                pltpu.sync_copy(stage, acc.at[pl.ds(j * _ZCH, _ZCH)])

    def _scatter():
        def chunk(i, carry):
            base = s * _ES2 + i * _CH
            pltpu.sync_copy(col_hbm.at[pl.ds(base, _CH)], idx)
            pltpu.sync_copy(en_hbm.at[pl.ds(base, _CH)], rows)
            for j in range(_CH // 16):
                v = idx[pl.ds(16 * j, 16)] - lo
                ok = (v >= 0) & (v < _NR)
                idx[pl.ds(16 * j, 16)] = jnp.where(ok, v, _NR)
            pltpu.sync_copy(rows, acc.at[idx], add=True)
            return carry
        lax.fori_loop(0, _NCH2, chunk, 0)

    def _writeout(dst_hbm):
        for k in range(-(-_NW0 // _NS)):
            j = s + k * _NS

            @pl.when(j < _NW0)
            def _():
                pltpu.sync_copy(acc.at[pl.ds(j * _ZCH, _ZCH)], stage)
                pltpu.sync_copy(stage, dst_hbm.at[pl.ds(lo + j * _ZCH, _ZCH)])

    _fill_stage(z16)
    _zero_acc()
    plsc.subcore_barrier()
    _scatter()
    plsc.subcore_barrier()
    _writeout(agg_hbm)


def _sc_scatter(e_new, col):
    mesh = plsc.VectorSubcoreMesh(core_axis_name="c", subcore_axis_name="s")
    fn = pl.kernel(
        _scatter_body,
        out_type=jax.ShapeDtypeStruct((N, H), F32),
        mesh=mesh,
        scratch_types=[
            pltpu.VMEM((_CH,), jnp.int32),
            pltpu.VMEM((_CH, H), F32),
            pltpu.VMEM((_ZCH, H), F32),
            pltpu.VMEM_SHARED((_NRP, H), F32),
        ],
    )
    return fn(e_new, col)


_BN = 2000
_NBN = N // _BN


def _node_body(x_ref, agg_ref, cum_ref, b2d_ref, up_ref,
               lng_ref, lnb_ref, mkT_ref, mv_ref,
               nw1T_ref, nb1_ref, nw2T_ref, nb2_ref,
               g1uT_ref, g1nT_ref, g1eT_ref, gb1_ref, gw2T_ref, gb2_ref,
               xn_ref, un_ref,
               ns_ref, nc_ref, es_ref):
    i = pl.program_id(0)

    @pl.when(i == 0)
    def _():
        ns_ref[...] = jnp.zeros_like(ns_ref)
        nc_ref[...] = jnp.zeros_like(nc_ref)
        es_ref[...] = jnp.zeros_like(es_ref)

    xb = x_ref[...]
    aggb = agg_ref[...]
    oh = (b2d_ref[...] == lax.broadcasted_iota(jnp.int32, (_BN, B), 1)
          ).astype(F32)
    ub = _dot(oh, up_ref[...])
    out = jnp.concatenate([xb, aggb, ub], axis=1)
    s1 = jnp.sum(out, axis=1, keepdims=True)
    mu = s1 * (1.0 / D_NODE)
    s2 = jnp.sum(out * out, axis=1, keepdims=True)
    var = s2 * (1.0 / D_NODE) - mu * mu
    r = lax.rsqrt(var + 1e-5)
    outn = (out - mu) * r * lng_ref[...] + lnb_ref[...]
    sc = _dot(outn, mkT_ref[...])
    m = jnp.max(sc, axis=1, keepdims=True)
    ex = jnp.exp(sc - m)
    p = ex / jnp.sum(ex, axis=1, keepdims=True)
    attn = _dot(p, mv_ref[...])
    out2 = 0.5 * attn + 0.5 * outn
    hn = jnp.maximum(_dot(out2, nw1T_ref[...]) + nb1_ref[...], 0.0)
    xn = _dot(hn, nw2T_ref[...]) + nb2_ref[...]
    xn_ref[...] = xn

    ones = jnp.ones((_BN, H), F32)
    ns_ref[...] += lax.dot_general(oh, xn, _DN0, preferred_element_type=F32)
    nc_ref[...] += lax.dot_general(oh, ones, _DN0, preferred_element_type=F32)
    es_ref[...] += lax.dot_general(oh, aggb, _DN0, preferred_element_type=F32)

    @pl.when(i == _NBN - 1)
    def _():
        cum = cum_ref[...]
        i0 = lax.broadcasted_iota(jnp.int32, (B, B), 0)
        i1 = lax.broadcasted_iota(jnp.int32, (B, B), 1)
        dm = (i1 == i0).astype(F32) - (i1 == i0 + 1).astype(F32)
        ec = lax.dot_general(dm, cum, (((1,), (1,)), ((), ())),
                             preferred_element_type=F32)
        na = ns_ref[...] / jnp.maximum(nc_ref[...], 1.0)
        eg = es_ref[...] / jnp.maximum(ec, 1.0)
        hg = jnp.maximum(_dot(up_ref[...], g1uT_ref[...]) + _dot(na, g1nT_ref[...])
                         + _dot(eg, g1eT_ref[...]) + gb1_ref[...], 0.0)
        un_ref[...] = _dot(hg, gw2T_ref[...]) + gb2_ref[...]


def _node_global(x, agg, cum, batch2d, u_pad, lng_pad, lnb_pad, mkT_pad,
                 mv_pad, nw1T_pad, nb1row, nw2T, nb2row,
                 g1uT_pad, g1nT, g1eT, gb1row, gw2T, gb2row):
    cw = lambda shape: pl.BlockSpec(shape, lambda i: tuple(0 for _ in shape))
    return pl.pallas_call(
        _node_body,
        grid=(_NBN,),
        in_specs=[
            pl.BlockSpec((_BN, H), lambda i: (i, 0)),
            pl.BlockSpec((_BN, H), lambda i: (i, 0)),
            cw((1, B)),
            pl.BlockSpec((_BN, 1), lambda i: (i, 0)),
            cw((B, H)),
            cw((1, D_PAD)), cw((1, D_PAD)),
            cw((D_PAD, H)), cw((H, D_PAD)),
            cw((D_PAD, H)), cw((1, H)), cw((H, H)), cw((1, H)),
            cw((H, H)), cw((H, H)), cw((H, H)), cw((1, H)),
            cw((H, H)), cw((1, H)),
        ],
        out_specs=(
            pl.BlockSpec((_BN, H), lambda i: (i, 0)),
            pl.BlockSpec((B, H), lambda i: (0, 0)),
        ),
        out_shape=(jax.ShapeDtypeStruct((N, H), F32),
                   jax.ShapeDtypeStruct((B, H), F32)),
        scratch_shapes=[
            pltpu.VMEM((B, H), F32),
            pltpu.VMEM((B, H), F32),
            pltpu.VMEM((B, H), F32),
        ],
    )(x, agg, cum, batch2d, u_pad, lng_pad, lnb_pad, mkT_pad, mv_pad,
      nw1T_pad, nb1row, nw2T, nb2row, g1uT_pad, g1nT, g1eT, gb1row, gw2T,
      gb2row)


def kernel(x, edge_index, edge_attr, u, batch,
           e_w1, e_b1, e_w2, e_b2, ln_g, ln_b, Mk, Mv,
           n_w1, n_b1, n_w2, n_b2, g_w1, g_b1, g_w2, g_b2):
    row = edge_index[0]
    col = edge_index[1]
    batch2d = batch.reshape(N, 1)

    w1aT = e_w1[:, :V].T
    w1bT = e_w1[:, V:2 * V].T
    w1cT = e_w1[:, 2 * V:2 * V + EIN].T
    w1dT_pad = jnp.zeros((H, H), F32).at[:U].set(e_w1[:, 2 * V + EIN:].T)
    u_pad = jnp.zeros((B, H), F32).at[:, :U].set(u)
    b1row = e_b1.reshape(1, H)
    w2T = e_w2.T
    b2row = e_b2.reshape(1, H)

    lng_pad = jnp.zeros((1, D_PAD), F32).at[0, :D_NODE].set(ln_g)
    lnb_pad = jnp.zeros((1, D_PAD), F32).at[0, :D_NODE].set(ln_b)
    mkT_pad = jnp.zeros((D_PAD, H), F32).at[:D_NODE].set(Mk.T)
    mv_pad = jnp.zeros((H, D_PAD), F32).at[:, :D_NODE].set(Mv)
    nw1T_pad = jnp.zeros((D_PAD, H), F32).at[:D_NODE].set(n_w1.T)
    nb1row = n_b1.reshape(1, H)
    nw2T = n_w2.T
    nb2row = n_b2.reshape(1, H)
    g1uT_pad = jnp.zeros((H, H), F32).at[:U].set(g_w1[:, :U].T)
    g1nT = g_w1[:, U:U + H].T
    g1eT = g_w1[:, U + H:].T
    gb1row = g_b1.reshape(1, H)
    gw2T = g_w2.T
    gb2row = g_b2.reshape(1, H)

    xa2, xb2, starts = _precompute(x, u_pad, batch2d, w1aT, w1bT, w1dT_pad,
                                   b1row)
    xr, xc = _sc_gather(xa2, xb2, row, col)
    e_new, cum = _edge_mlp(xr, xc, edge_attr, col.reshape(E, 1), starts,
                           w1cT, w2T, b2row)
    agg = _sc_scatter(e_new, col)
    x_new, u_new = _node_global(
        x, agg, cum, batch2d, u_pad,
        lng_pad, lnb_pad, mkT_pad, mv_pad, nw1T_pad, nb1row, nw2T, nb2row,
        g1uT_pad, g1nT, g1eT, gb1row, gw2T, gb2row)
    return (x_new, e_new, u_new)
